# baseline (speedup 1.0000x reference)
import functools

import jax
import jax.numpy as jnp
from jax import lax
from jax.experimental import pallas as pl
from jax.experimental.pallas import tpu as pltpu
from jax.experimental.pallas import tpu_sc as plsc

_NCLS = 10
_N = 16384
_L = 16
_NW = 16
_RPW = _N // _NW
_GROUPS = _RPW // _L
_LN2 = 0.6931471805599453
_SQRT2 = 1.4142135623730951


def _vlog(x):
    bits = plsc.bitcast(x, jnp.int32)
    e = lax.shift_right_arithmetic(bits, 23) - 127
    m = plsc.bitcast((bits & 0x007FFFFF) | 0x3F800000, jnp.float32)
    big = m >= _SQRT2
    m2 = jnp.where(big, m * 0.5, m)
    ef = e.astype(jnp.float32)
    e2 = jnp.where(big, ef + 1.0, ef)
    t = (m2 - 1.0) / (m2 + 1.0)
    t2 = t * t
    p = t * (2.0 + t2 * (2.0 / 3.0 + t2 * (2.0 / 5.0 + t2 * (2.0 / 7.0 + t2 * (2.0 / 9.0)))))
    return e2 * _LN2 + p


def _body(x_hbm, lab_hbm, prior_hbm, colt_hbm, out_hbm, pub_hbm,
          x_v, lab_v, prior_v, colt_v, part_v, red_v, out_v,
          len_idx):
    w = lax.axis_index("s")
    pltpu.sync_copy(x_hbm.at[pl.ds(w * _RPW, _RPW), :], x_v)
    pltpu.sync_copy(lab_hbm.at[w], lab_v)
    pltpu.sync_copy(prior_hbm, prior_v)
    pltpu.sync_copy(colt_hbm, colt_v)

    lanes = lax.iota(jnp.int32, _L)
    zf = jnp.zeros((_L,), jnp.float32)
    one = zf + 1.0

    for r in range(_L):
        part_v[r, :] = zf

    def grp(g, carry):
        lab = lab_v[pl.ds(g * _L, _L)]
        pm = lab <= _NCLS - 1
        um = jnp.logical_not(pm)
        rows = g * _L + lanes
        xcs = [plsc.load_gather(x_v, [rows, colt_v[pl.ds(c * _L, _L)]])
               for c in range(_NCLS)]
        mx = xcs[0]
        for c in range(1, _NCLS):
            mx = jnp.maximum(mx, xcs[c])
        es = [jnp.exp(xc - mx) for xc in xcs]
        s = es[0]
        for c in range(1, _NCLS):
            s = s + es[c]
        rinv = 1.0 / s
        ln_s = _vlog(s)
        pu2 = zf
        xsel = zf
        for c in range(_NCLS):
            p_c = es[c] * rinv
            t_c = -_vlog(1.01 - p_c)
            mcl = um if c < len_idx else pm
            part_v[c, :] = part_v[c, :] + jnp.where(mcl, t_c, zf)
            sel = pm & (lab == c)
            pu2 = pu2 + jnp.where(sel, t_c * prior_v[c, :], zf)
            xsel = xsel + jnp.where(sel, xcs[c], zf)
        part_v[10, :] = part_v[10, :] + pu2
        part_v[11, :] = part_v[11, :] + xsel
        part_v[12, :] = part_v[12, :] + jnp.where(pm, mx + ln_s, zf)
        part_v[13, :] = part_v[13, :] + jnp.where(pm, one, zf)
        part_v[14, :] = part_v[14, :] + jnp.where(um, one, zf)
        return carry

    lax.fori_loop(0, _GROUPS, grp, jnp.int32(0))

    pv = zf
    for r in range(15):
        pv = jnp.where(lanes == r + 1, zf + jnp.sum(part_v[r, :]), pv)
    part_v[0, :] = pv
    pltpu.sync_copy(part_v.at[0], pub_hbm.at[w])
    plsc.subcore_barrier()

    @pl.when(w == 0)
    def _():
        pltpu.sync_copy(pub_hbm, red_v)
        totv = red_v[0, :]
        for j in range(1, _NW):
            totv = totv + red_v[j, :]
        out_v[...] = totv
        tb = [plsc.load_gather(out_v, [lanes * 0 + (r + 1)]) for r in range(15)]
        prior0_b = prior_v[10, :]
        n_p_v = tb[13]
        n_u_v = tb[14]
        mx_p_v = jnp.maximum(n_p_v, one)
        mx_u_v = jnp.maximum(n_u_v, one)
        w_u_v = one / (mx_u_v * float(len_idx))
        w_p = prior0_b / (mx_p_v * float(_NCLS - len_idx))
        pc = zf
        for c in range(_NCLS):
            pc = pc + tb[c] * (w_u_v if c < len_idx else w_p)
        pul_v = pc - tb[10] / mx_p_v
        cl_v = (tb[12] - tb[11]) / n_p_v
        ov = jnp.where(lanes == 1, pul_v, cl_v)
        ov = jnp.where(lanes >= 3, zf, ov)
        out_v[...] = ov
        pltpu.sync_copy(out_v, out_hbm)


def kernel(outputs, labels, priorlist, indexlist):
    len_idx = int(indexlist.shape[0])
    x = outputs.astype(jnp.float32)
    lab = labels.astype(jnp.int32).reshape(_NW, _RPW)
    pf = priorlist.astype(jnp.float32)
    prior_bc = jnp.concatenate([pf, pf[indexlist[0]][None]])
    prior_bc = jnp.broadcast_to(prior_bc[:, None], (_NCLS + 1, _L))
    colt = jnp.broadcast_to(
        jnp.arange(_NCLS, dtype=jnp.int32)[:, None], (_NCLS, _L)).reshape(-1)
    mesh = plsc.VectorSubcoreMesh(
        core_axis_name="c", subcore_axis_name="s", num_cores=1,
        num_subcores=_NW)
    out, _ = pl.kernel(
        functools.partial(_body, len_idx=len_idx),
        out_type=(jax.ShapeDtypeStruct((_L,), jnp.float32),
                  jax.ShapeDtypeStruct((_NW, _L), jnp.float32)),
        mesh=mesh,
        compiler_params=pltpu.CompilerParams(
            needs_layout_passes=False, use_tc_tiling_on_sc=False),
        scratch_types=[
            pltpu.VMEM((_RPW, _NCLS), jnp.float32),
            pltpu.VMEM((_RPW,), jnp.int32),
            pltpu.VMEM((_NCLS + 1, _L), jnp.float32),
            pltpu.VMEM((_NCLS * _L,), jnp.int32),
            pltpu.VMEM((_L, _L), jnp.float32),
            pltpu.VMEM((_NW, _L), jnp.float32),
            pltpu.VMEM((_L,), jnp.float32),
        ],
    )(x, lab, prior_bc, colt)
    return (out[0], out[1:2], out[2])


# speedup vs baseline: 1.1224x; 1.1224x over previous
import functools

import jax
import jax.numpy as jnp
from jax import lax
from jax.experimental import pallas as pl
from jax.experimental.pallas import tpu as pltpu
from jax.experimental.pallas import tpu_sc as plsc

_NCLS = 10
_N = 16384
_L = 16
_NW = 16
_RPW = _N // _NW
_GROUPS = _RPW // _L
_LN2 = 0.6931471805599453
_SQRT2 = 1.4142135623730951


def _vlog(x):
    bits = plsc.bitcast(x, jnp.int32)
    e = lax.shift_right_arithmetic(bits, 23) - 127
    m = plsc.bitcast((bits & 0x007FFFFF) | 0x3F800000, jnp.float32)
    big = m >= _SQRT2
    m2 = jnp.where(big, m * 0.5, m)
    ef = e.astype(jnp.float32)
    e2 = jnp.where(big, ef + 1.0, ef)
    t = (m2 - 1.0) / (m2 + 1.0)
    t2 = t * t
    p = t * (2.0 + t2 * (2.0 / 3.0 + t2 * (2.0 / 5.0 + t2 * (2.0 / 7.0))))
    return e2 * _LN2 + p


def _body(x_hbm, lab_hbm, prior_hbm, out_hbm, pub_hbm,
          x_v, lab_v, prior_v, part_v, red_v, out_v,
          len_idx):
    w = lax.axis_index("s")
    pltpu.sync_copy(x_hbm.at[pl.ds(w * (_RPW * _NCLS), _RPW * _NCLS)], x_v)
    pltpu.sync_copy(lab_hbm.at[pl.ds(w * _RPW, _RPW)], lab_v)
    pltpu.sync_copy(prior_hbm, prior_v)

    lanes = lax.iota(jnp.int32, _L)
    zf = jnp.zeros((_L,), jnp.float32)
    one = zf + 1.0

    for r in range(_L):
        part_v[r, :] = zf

    def grp(g, carry):
        lab = lab_v[pl.ds(g * _L, _L)]
        pm = lab <= _NCLS - 1
        um = jnp.logical_not(pm)
        base = g * (_L * _NCLS)
        xcs = [plsc.load_gather(x_v, [base + lanes * _NCLS + c])
               for c in range(_NCLS)]
        mx = xcs[0]
        for c in range(1, _NCLS):
            mx = jnp.maximum(mx, xcs[c])
        es = [jnp.exp(xc - mx) for xc in xcs]
        s = es[0]
        for c in range(1, _NCLS):
            s = s + es[c]
        rinv = 1.0 / s
        ln_s = _vlog(s)
        pu2 = zf
        xsel = zf
        for c in range(_NCLS):
            p_c = es[c] * rinv
            t_c = -_vlog(1.01 - p_c)
            mcl = um if c < len_idx else pm
            part_v[c, :] = part_v[c, :] + jnp.where(mcl, t_c, zf)
            sel = pm & (lab == c)
            pu2 = pu2 + jnp.where(sel, t_c * prior_v[c, :], zf)
            xsel = xsel + jnp.where(sel, xcs[c], zf)
        part_v[10, :] = part_v[10, :] + pu2
        part_v[11, :] = part_v[11, :] + xsel
        part_v[12, :] = part_v[12, :] + jnp.where(pm, mx + ln_s, zf)
        part_v[13, :] = part_v[13, :] + jnp.where(pm, one, zf)
        part_v[14, :] = part_v[14, :] + jnp.where(um, one, zf)
        return carry

    lax.fori_loop(0, _GROUPS, grp, jnp.int32(0))

    pv = zf
    for r in range(15):
        pv = jnp.where(lanes == r + 1, zf + jnp.sum(part_v[r, :]), pv)
    part_v[0, :] = pv
    pltpu.sync_copy(part_v.at[0], pub_hbm.at[w])
    plsc.subcore_barrier()

    @pl.when(w == 0)
    def _():
        pltpu.sync_copy(pub_hbm, red_v)
        totv = red_v[0, :]
        for j in range(1, _NW):
            totv = totv + red_v[j, :]
        out_v[...] = totv
        tb = [plsc.load_gather(out_v, [lanes * 0 + (r + 1)]) for r in range(15)]
        prior0_b = prior_v[10, :]
        n_p_v = tb[13]
        n_u_v = tb[14]
        mx_p_v = jnp.maximum(n_p_v, one)
        mx_u_v = jnp.maximum(n_u_v, one)
        w_u_v = one / (mx_u_v * float(len_idx))
        w_p = prior0_b / (mx_p_v * float(_NCLS - len_idx))
        pc = zf
        for c in range(_NCLS):
            pc = pc + tb[c] * (w_u_v if c < len_idx else w_p)
        pul_v = pc - tb[10] / mx_p_v
        cl_v = (tb[12] - tb[11]) / n_p_v
        ov = jnp.where(lanes == 1, pul_v, cl_v)
        ov = jnp.where(lanes >= 3, zf, ov)
        out_v[...] = ov
        pltpu.sync_copy(out_v, out_hbm)


def kernel(outputs, labels, priorlist, indexlist):
    len_idx = int(indexlist.shape[0])
    x = outputs.astype(jnp.float32).reshape(-1)
    lab = labels.astype(jnp.int32)
    pf = priorlist.astype(jnp.float32)
    prior_bc = jnp.concatenate([pf, pf[indexlist[0]][None]])
    prior_bc = jnp.broadcast_to(prior_bc[:, None], (_NCLS + 1, _L))
    mesh = plsc.VectorSubcoreMesh(
        core_axis_name="c", subcore_axis_name="s", num_cores=1,
        num_subcores=_NW)
    out, _ = pl.kernel(
        functools.partial(_body, len_idx=len_idx),
        out_type=(jax.ShapeDtypeStruct((_L,), jnp.float32),
                  jax.ShapeDtypeStruct((_NW, _L), jnp.float32)),
        mesh=mesh,
        compiler_params=pltpu.CompilerParams(needs_layout_passes=False),
        scratch_types=[
            pltpu.VMEM((_RPW * _NCLS,), jnp.float32),
            pltpu.VMEM((_RPW,), jnp.int32),
            pltpu.VMEM((_NCLS + 1, _L), jnp.float32),
            pltpu.VMEM((_L, _L), jnp.float32),
            pltpu.VMEM((_NW, _L), jnp.float32),
            pltpu.VMEM((_L,), jnp.float32),
        ],
    )(x, lab, prior_bc)
    return (out[0], out[1:2], out[2])


# speedup vs baseline: 1.1473x; 1.0222x over previous
import functools

import jax
import jax.numpy as jnp
from jax import lax
from jax.experimental import pallas as pl
from jax.experimental.pallas import tpu as pltpu
from jax.experimental.pallas import tpu_sc as plsc

_NCLS = 10
_N = 16384
_L = 16
_NW = 16
_RPW = _N // _NW
_GROUPS = _RPW // _L
_LN2 = 0.6931471805599453
_SQRT2 = 1.4142135623730951


def _vlog(x):
    bits = plsc.bitcast(x, jnp.int32)
    e = lax.shift_right_arithmetic(bits, 23) - 127
    m = plsc.bitcast((bits & 0x007FFFFF) | 0x3F800000, jnp.float32)
    big = m >= _SQRT2
    m2 = jnp.where(big, m * 0.5, m)
    ef = e.astype(jnp.float32)
    e2 = jnp.where(big, ef + 1.0, ef)
    t = (m2 - 1.0) / (m2 + 1.0)
    t2 = t * t
    p = t * (2.0 + t2 * (2.0 / 3.0 + t2 * (2.0 / 5.0 + t2 * (2.0 / 7.0))))
    return e2 * _LN2 + p


def _body(x_hbm, lab_hbm, prior_hbm, out_hbm, pub_hbm,
          x_v, lab_v, prior_v, part_v, red_v, out_v,
          len_idx):
    w = lax.axis_index("s")
    pltpu.sync_copy(x_hbm.at[pl.ds(w * (_RPW * _NCLS), _RPW * _NCLS)], x_v)
    pltpu.sync_copy(lab_hbm.at[pl.ds(w * _RPW, _RPW)], lab_v)
    pltpu.sync_copy(prior_hbm, prior_v)

    lanes = lax.iota(jnp.int32, _L)
    zf = jnp.zeros((_L,), jnp.float32)
    one = zf + 1.0

    for r in range(_L):
        part_v[r, :] = zf

    def grp(g, carry):
        labs, pms, ums, tcs, xss, mss = [], [], [], [], [], []
        for h in range(2):
            gg = g * 2 + h
            lab = lab_v[pl.ds(gg * _L, _L)]
            pm = lab <= _NCLS - 1
            um = jnp.logical_not(pm)
            base = gg * (_L * _NCLS)
            xcs = [plsc.load_gather(x_v, [base + lanes * _NCLS + c])
                   for c in range(_NCLS)]
            mx = xcs[0]
            for c in range(1, _NCLS):
                mx = jnp.maximum(mx, xcs[c])
            es = [jnp.exp(xc - mx) for xc in xcs]
            s = es[0]
            for c in range(1, _NCLS):
                s = s + es[c]
            rinv = 1.0 / s
            ln_s = _vlog(s)
            tcs.append([-_vlog(1.01 - es[c] * rinv) for c in range(_NCLS)])
            labs.append(lab)
            pms.append(pm)
            ums.append(um)
            xss.append(xcs)
            mss.append(mx + ln_s)
        pu2 = zf
        xsel = zf
        for c in range(_NCLS):
            acc = part_v[c, :]
            for h in range(2):
                mcl = ums[h] if c < len_idx else pms[h]
                acc = acc + jnp.where(mcl, tcs[h][c], zf)
                sel = pms[h] & (labs[h] == c)
                pu2 = pu2 + jnp.where(sel, tcs[h][c] * prior_v[c, :], zf)
                xsel = xsel + jnp.where(sel, xss[h][c], zf)
            part_v[c, :] = acc
        part_v[10, :] = part_v[10, :] + pu2
        part_v[11, :] = part_v[11, :] + xsel
        part_v[12, :] = (part_v[12, :]
                         + jnp.where(pms[0], mss[0], zf)
                         + jnp.where(pms[1], mss[1], zf))
        part_v[13, :] = (part_v[13, :]
                         + jnp.where(pms[0], one, zf)
                         + jnp.where(pms[1], one, zf))
        part_v[14, :] = (part_v[14, :]
                         + jnp.where(ums[0], one, zf)
                         + jnp.where(ums[1], one, zf))
        return carry

    lax.fori_loop(0, _GROUPS // 2, grp, jnp.int32(0))

    pv = zf
    for r in range(15):
        pv = jnp.where(lanes == r + 1, zf + jnp.sum(part_v[r, :]), pv)
    part_v[0, :] = pv
    pltpu.sync_copy(part_v.at[0], pub_hbm.at[w])
    plsc.subcore_barrier()

    @pl.when(w == 0)
    def _():
        pltpu.sync_copy(pub_hbm, red_v)
        totv = red_v[0, :]
        for j in range(1, _NW):
            totv = totv + red_v[j, :]
        out_v[...] = totv
        tb = [plsc.load_gather(out_v, [lanes * 0 + (r + 1)]) for r in range(15)]
        prior0_b = prior_v[10, :]
        n_p_v = tb[13]
        n_u_v = tb[14]
        mx_p_v = jnp.maximum(n_p_v, one)
        mx_u_v = jnp.maximum(n_u_v, one)
        w_u_v = one / (mx_u_v * float(len_idx))
        w_p = prior0_b / (mx_p_v * float(_NCLS - len_idx))
        pc = zf
        for c in range(_NCLS):
            pc = pc + tb[c] * (w_u_v if c < len_idx else w_p)
        pul_v = pc - tb[10] / mx_p_v
        cl_v = (tb[12] - tb[11]) / n_p_v
        ov = jnp.where(lanes == 1, pul_v, cl_v)
        ov = jnp.where(lanes >= 3, zf, ov)
        out_v[...] = ov
        pltpu.sync_copy(out_v, out_hbm)


def kernel(outputs, labels, priorlist, indexlist):
    len_idx = int(indexlist.shape[0])
    x = outputs.astype(jnp.float32).reshape(-1)
    lab = labels.astype(jnp.int32)
    pf = priorlist.astype(jnp.float32)
    prior_bc = jnp.concatenate([pf, pf[indexlist[0]][None]])
    prior_bc = jnp.broadcast_to(prior_bc[:, None], (_NCLS + 1, _L))
    mesh = plsc.VectorSubcoreMesh(
        core_axis_name="c", subcore_axis_name="s", num_cores=1,
        num_subcores=_NW)
    out, _ = pl.kernel(
        functools.partial(_body, len_idx=len_idx),
        out_type=(jax.ShapeDtypeStruct((_L,), jnp.float32),
                  jax.ShapeDtypeStruct((_NW, _L), jnp.float32)),
        mesh=mesh,
        compiler_params=pltpu.CompilerParams(needs_layout_passes=False),
        scratch_types=[
            pltpu.VMEM((_RPW * _NCLS,), jnp.float32),
            pltpu.VMEM((_RPW,), jnp.int32),
            pltpu.VMEM((_NCLS + 1, _L), jnp.float32),
            pltpu.VMEM((_L, _L), jnp.float32),
            pltpu.VMEM((_NW, _L), jnp.float32),
            pltpu.VMEM((_L,), jnp.float32),
        ],
    )(x, lab, prior_bc)
    return (out[0], out[1:2], out[2])
